# baseline (speedup 1.0000x reference)
import functools

import jax
import jax.numpy as jnp
import numpy as np
from jax.experimental import pallas as pl
from jax.experimental.pallas import tpu as pltpu

SH_C0 = 0.28209479177387814
SH_C1 = 0.4886025119029199
SH_C2 = (1.0925484305920792, -1.0925484305920792, 0.31539156525252005, -1.0925484305920792, 0.5462742152960396)
SH_C3 = (-0.5900435899266435, 2.890611442640554, -0.4570457994644658, 0.3731763325901154, -0.4570457994644658, 1.445305721320277, -0.5900435899266435)

_W = 128
_H = 128
_NPIX = 512
_CH = 256


def _eval_sh3(sh, dirs):
    x = dirs[:, 0:1]; y = dirs[:, 1:2]; z = dirs[:, 2:3]
    result = SH_C0 * sh[:, 0]
    result = result - SH_C1 * y * sh[:, 1] + SH_C1 * z * sh[:, 2] - SH_C1 * x * sh[:, 3]
    xx = x * x; yy = y * y; zz = z * z
    xy = x * y; yz = y * z; xz = x * z
    result = (result + SH_C2[0] * xy * sh[:, 4] + SH_C2[1] * yz * sh[:, 5]
              + SH_C2[2] * (2.0 * zz - xx - yy) * sh[:, 6]
              + SH_C2[3] * xz * sh[:, 7] + SH_C2[4] * (xx - yy) * sh[:, 8])
    result = (result + SH_C3[0] * y * (3.0 * xx - yy) * sh[:, 9]
              + SH_C3[1] * xy * z * sh[:, 10]
              + SH_C3[2] * y * (4.0 * zz - xx - yy) * sh[:, 11]
              + SH_C3[3] * z * (2.0 * zz - 3.0 * xx - 3.0 * yy) * sh[:, 12]
              + SH_C3[4] * x * (4.0 * zz - xx - yy) * sh[:, 13]
              + SH_C3[5] * z * (xx - yy) * sh[:, 14]
              + SH_C3[6] * x * (xx - 3.0 * yy) * sh[:, 15])
    return jnp.maximum(result + 0.5, 0.0)


def _quat_to_rot(q):
    q = q / (jnp.linalg.norm(q, axis=1, keepdims=True) + 1e-8)
    r = q[:, 0]; x = q[:, 1]; y = q[:, 2]; z = q[:, 3]
    R = jnp.stack([
        1 - 2 * (y * y + z * z), 2 * (x * y - r * z), 2 * (x * z + r * y),
        2 * (x * y + r * z), 1 - 2 * (x * x + z * z), 2 * (y * z - r * x),
        2 * (x * z - r * y), 2 * (y * z + r * x), 1 - 2 * (x * x + y * y)], axis=1)
    return R.reshape(-1, 3, 3)


def _prep_body(pre7, g3, shsr, parr, attrs, aux):
    pxr = pre7[0]; pyr = pre7[1]
    c0 = pre7[2]; c1 = pre7[3]; c2 = pre7[4]
    op_eff = pre7[5]; depth = pre7[6]
    mx = g3[0]; my = g3[1]; mz = g3[2]
    cx = parr[0]; cy = parr[1]; cz = parr[2]
    ddx = mx - cx; ddy = my - cy; ddz = mz - cz
    di = 1.0 / (jnp.sqrt(ddx * ddx + ddy * ddy + ddz * ddz) + 1e-8)
    dxn = ddx * di; dyn = ddy * di; dzn = ddz * di
    xx = dxn * dxn; yy = dyn * dyn; zz = dzn * dzn
    xy = dxn * dyn; yz = dyn * dzn; xz = dxn * dzn
    bas = [None] * 16
    bas[1] = -SH_C1 * dyn; bas[2] = SH_C1 * dzn; bas[3] = -SH_C1 * dxn
    bas[4] = SH_C2[0] * xy; bas[5] = SH_C2[1] * yz
    bas[6] = SH_C2[2] * (2.0 * zz - xx - yy)
    bas[7] = SH_C2[3] * xz; bas[8] = SH_C2[4] * (xx - yy)
    bas[9] = SH_C3[0] * dyn * (3.0 * xx - yy)
    bas[10] = SH_C3[1] * xy * dzn
    bas[11] = SH_C3[2] * dyn * (4.0 * zz - xx - yy)
    bas[12] = SH_C3[3] * dzn * (2.0 * zz - 3.0 * xx - 3.0 * yy)
    bas[13] = SH_C3[4] * dxn * (4.0 * zz - xx - yy)
    bas[14] = SH_C3[5] * dzn * (xx - yy)
    bas[15] = SH_C3[6] * dxn * (xx - 3.0 * yy)
    rgb = []
    for ch in range(3):
        col = SH_C0 * shsr[ch]
        for i in range(1, 16):
            col = col + bas[i] * shsr[3 * i + ch]
        rgb.append(jnp.maximum(col + 0.5, 0.0))
    midc = 0.5 * (c0 + c2)
    detc = c0 * c2 - c1 * c1
    lminc = midc - jnp.sqrt(jnp.maximum(midc * midc - detc, 0.0))
    thresh = 2.0 * jnp.log(255.0 * jnp.maximum(op_eff, 1e-30))
    radc = jnp.sqrt(jnp.maximum(thresh, 0.0)
                    / jnp.maximum(lminc, 1e-30)) * 1.001 + 0.1
    okf = (op_eff > 0.0) & (thresh > 0.0) & (lminc > 0.0)
    badf = jnp.where(okf & jnp.logical_not(radc <= jnp.float32(_TILE)), 1.0, 0.0)
    inv_t = 1.0 / _TILE
    g = jnp.float32(_TGRID - 1)
    tx0 = jnp.clip(jnp.floor((pxr - radc) * inv_t), 0.0, g)
    tx1 = jnp.clip(jnp.floor((pxr + radc) * inv_t), 0.0, g)
    ty0 = jnp.clip(jnp.floor((pyr - radc) * inv_t), 0.0, g)
    ty1 = jnp.clip(jnp.floor((pyr + radc) * inv_t), 0.0, g)
    one = jnp.full_like(pxr, 1.0)
    for i, row in enumerate([pxr, pyr, c0, c1, c2, op_eff, rgb[0], rgb[1],
                             rgb[2], depth, one, tx0, tx1, ty0, ty1,
                             jnp.where(okf, 1.0, 0.0)]):
        attrs[i] = row
    aux[0] = badf


def _preprocess_nosh(width, height, means3D, opacities, scales, scale_modifier,
                     rotations, viewmatrix, projmatrix, tanfovx, tanfovy):
    P = means3D.shape[0]
    ones_col = jnp.ones((P, 1), dtype=jnp.float32)
    means_hom = jnp.concatenate([means3D, ones_col], axis=1)
    vm = viewmatrix.astype(jnp.float32)
    pm = projmatrix.astype(jnp.float32)
    focal_y = height / (2.0 * tanfovy)
    focal_x = width / (2.0 * tanfovx)
    p_view = means_hom @ vm
    depths = p_view[:, 2]
    p_hom = means_hom @ pm
    p_w = 1.0 / (p_hom[:, 3:4] + 1e-7)
    p_proj = p_hom[:, :3] * p_w
    Rm = _quat_to_rot(rotations)
    s = scales * scale_modifier
    Sigma = jnp.einsum('pij,pj,pkj->pik', Rm, s * s, Rm)
    t = p_view[:, :3]
    tz = t[:, 2]
    limx = 1.3 * tanfovx
    limy = 1.3 * tanfovy
    tx = jnp.clip(t[:, 0] / tz, -limx, limx) * tz
    ty = jnp.clip(t[:, 1] / tz, -limy, limy) * tz
    zero = jnp.zeros_like(tz)
    J0 = jnp.stack([focal_x / tz, zero, -focal_x * tx / (tz * tz)], axis=1)
    J1 = jnp.stack([zero, focal_y / tz, -focal_y * ty / (tz * tz)], axis=1)
    J = jnp.stack([J0, J1], axis=1)
    Wr = vm[:3, :3].T
    Tm = jnp.einsum('pij,jk->pik', J, Wr)
    cov2D = jnp.einsum('pij,pjk,plk->pil', Tm, Sigma, Tm)
    a = cov2D[:, 0, 0] + 0.3
    b = cov2D[:, 0, 1]
    c = cov2D[:, 1, 1] + 0.3
    det = a * c - b * b
    det_safe = jnp.where(jnp.abs(det) < 1e-12, 1.0, det)
    inv_det = 1.0 / det_safe
    conic = jnp.stack([c * inv_det, -b * inv_det, a * inv_det], axis=1)
    mid = 0.5 * (a + c)
    disc = jnp.sqrt(jnp.maximum(0.1, mid * mid - det))
    lam1 = mid + disc
    radius = jnp.ceil(3.0 * jnp.sqrt(jnp.maximum(lam1, 1e-8)))
    px = ((p_proj[:, 0] + 1.0) * width - 1.0) * 0.5
    py = ((p_proj[:, 1] + 1.0) * height - 1.0) * 0.5
    visible = (depths > 0.2) & (det > 0.0) & (radius > 0.0)
    radii = jnp.where(visible, radius, 0.0)
    return px, py, conic, opacities[:, 0], depths, visible, radii


def _keys_body(attrs_ref, keys_ref):
    tx0 = attrs_ref[11]; tx1 = attrs_ref[12]
    ty0 = attrs_ref[13]; ty1 = attrs_ref[14]
    ok = attrs_ref[15] > 0.5
    shp = tx0.shape
    rank = (jax.lax.broadcasted_iota(jnp.int32, shp, 0) * shp[1]
            + jax.lax.broadcasted_iota(jnp.int32, shp, 1))
    sent = jnp.int32(_NTILES << 13)
    for s in range(_KSLOT):
        txf = tx0 + jnp.float32(s % 3)
        tyf = ty0 + jnp.float32(s // 3)
        valid = ok & (txf <= tx1) & (tyf <= ty1)
        tile = (tyf * _TGRID + txf).astype(jnp.int32)
        keys_ref[s] = jnp.where(valid, (tile << 13) | rank, sent)


def _composite_body(attrs_ref, rgbd_ref, out_ref):
    i = pl.program_id(0)
    npix = _NPIX
    ch = _CH
    nchunks = attrs_ref.shape[1] // ch
    pix = i * npix + jax.lax.broadcasted_iota(jnp.int32, (npix, 1), 0)
    xf = (pix // _H).astype(jnp.float32)
    yf = (pix % _H).astype(jnp.float32)

    def shift_fill1(t, sh):
        return jnp.concatenate(
            [jnp.full((t.shape[0], sh), 1.0, t.dtype), t[:, :t.shape[1] - sh]],
            axis=1)

    def body(k, carry):
        tcar, acc = carry
        a = attrs_ref[:, pl.ds(k * ch, ch)]
        px_c = a[0:1, :]; py_c = a[1:2, :]
        c0 = a[2:3, :]; c1 = a[3:4, :]; c2 = a[4:5, :]
        opc = a[5:6, :]
        dx = px_c - xf
        dy = py_c - yf
        power = (-0.5 * (c0 * dx * dx + c2 * dy * dy)) - c1 * dx * dy
        alpha = opc * jnp.exp(jnp.minimum(power, 0.0))
        alpha = jnp.minimum(alpha, 0.99)
        alpha = jnp.where((power > 0.0) | (alpha < 1.0 / 255.0), 0.0, alpha)
        t = 1.0 - alpha
        sh = 1
        while sh < ch:
            t = t * shift_fill1(t, sh)
            sh *= 2
        tprev = tcar * shift_fill1(t, 1)
        w = jnp.where(tprev < 1e-4, 0.0, alpha * tprev)
        acc = acc + jax.lax.dot(w, rgbd_ref[pl.ds(k * ch, ch), :],
                                precision=jax.lax.Precision.HIGHEST)
        tcar = tcar * t[:, ch - 1:ch]
        return tcar, acc

    tcar0 = jnp.ones((npix, 1), jnp.float32)
    acc0 = jnp.zeros((npix, 8), jnp.float32)
    _, acc = jax.lax.fori_loop(0, nchunks, body, (tcar0, acc0))
    out_ref[...] = acc


def _composite(attrs, rgbd):
    npix_total = _W * _H
    grid = (npix_total // _NPIX,)
    return pl.pallas_call(
        _composite_body,
        grid=grid,
        in_specs=[
            pl.BlockSpec(attrs.shape, lambda i: (0, 0)),
            pl.BlockSpec(rgbd.shape, lambda i: (0, 0)),
        ],
        out_specs=pl.BlockSpec((_NPIX, 8), lambda i: (i, 0)),
        out_shape=jax.ShapeDtypeStruct((npix_total, 8), jnp.float32),
    )(attrs, rgbd)


_TILE = 16
_TGRID = _W // _TILE
_NTILES = _TGRID * _TGRID
_KSLOT = 9
_TPIX = _TILE * _TILE


def _tile_composite_body(starts_ref, binned_ref, out_ref):
    t = pl.program_id(0)
    start = starts_ref[t]
    end = starts_ref[t + 1]
    rr = jax.lax.broadcasted_iota(jnp.int32, (_TPIX, 1), 0)
    xf = ((t % _TGRID) * _TILE + rr // _TILE).astype(jnp.float32)
    yf = ((t // _TGRID) * _TILE + rr % _TILE).astype(jnp.float32)
    lane = jax.lax.broadcasted_iota(jnp.int32, (1, _CH), 1)

    def shift_fill1(v, sh):
        return jnp.concatenate(
            [jnp.full((v.shape[0], sh), 1.0, v.dtype), v[:, :v.shape[1] - sh]],
            axis=1)

    def chunk(j, carry):
        tcar, acc = carry
        a = binned_ref[j]
        o = j * _CH + lane
        valid = (o >= start) & (o < end)
        px_c = a[0:1, :]; py_c = a[1:2, :]
        c0 = a[2:3, :]; c1 = a[3:4, :]; c2 = a[4:5, :]
        opc = a[5:6, :]
        dx = px_c - xf
        dy = py_c - yf
        power = (-0.5 * (c0 * dx * dx + c2 * dy * dy)) - c1 * dx * dy
        alpha = opc * jnp.exp(jnp.minimum(power, 0.0))
        alpha = jnp.minimum(alpha, 0.99)
        alpha = jnp.where((power > 0.0) | (alpha < 1.0 / 255.0) | (~valid),
                          0.0, alpha)
        tv = 1.0 - alpha
        sh = 1
        while sh < _CH:
            tv = tv * shift_fill1(tv, sh)
            sh *= 2
        tprev = tcar * shift_fill1(tv, 1)
        w = jnp.where(tprev < 1e-4, 0.0, alpha * tprev)
        acc = acc + jax.lax.dot_general(
            w, a[6:14, :], (((1,), (1,)), ((), ())),
            precision=jax.lax.Precision.HIGHEST)
        tcar = tcar * tv[:, _CH - 1:_CH]
        return tcar, acc

    j0 = start // _CH
    j1 = (end + _CH - 1) // _CH
    tcar0 = jnp.ones((_TPIX, 1), jnp.float32)
    acc0 = jnp.zeros((_TPIX, 8), jnp.float32)
    _, acc = jax.lax.fori_loop(j0, j1, chunk, (tcar0, acc0))
    out_ref[0] = acc


def _tile_composite(starts, binned):
    grid_spec = pltpu.PrefetchScalarGridSpec(
        num_scalar_prefetch=1,
        grid=(_NTILES,),
        in_specs=[pl.BlockSpec(binned.shape, lambda t, s: (0, 0, 0))],
        out_specs=pl.BlockSpec((1, _TPIX, 8), lambda t, s: (t, 0, 0)),
    )
    return pl.pallas_call(
        _tile_composite_body,
        grid_spec=grid_spec,
        out_shape=jax.ShapeDtypeStruct((_NTILES, _TPIX, 8), jnp.float32),
    )(starts, binned)


def kernel(P, D, M, background, width, height, means3D, shs, opacities, scales,
           scale_modifier, rotations, viewmatrix, projmatrix, cam_pos,
           tanfovx, tanfovy):
    Pn = means3D.shape[0]
    sub = Pn // 1024
    px, py, conic, op, depths, visible, radii = _preprocess_nosh(
        width, height, means3D, opacities, scales, scale_modifier,
        rotations, viewmatrix, projmatrix, tanfovx, tanfovy)
    op_eff = jnp.where(visible, op, 0.0)
    pre7 = jnp.stack([px, py, conic[:, 0], conic[:, 1], conic[:, 2],
                      op_eff, depths]).reshape(7, sub, 1024)
    g3 = means3D.T.reshape(3, sub, 1024)
    shsr = shs.reshape(Pn, 48).T.reshape(48, sub, 1024)
    par = jnp.stack([cam_pos[0], cam_pos[1], cam_pos[2]])
    attrs_u, aux = pl.pallas_call(
        _prep_body,
        in_specs=[
            pl.BlockSpec(memory_space=pltpu.VMEM),
            pl.BlockSpec(memory_space=pltpu.VMEM),
            pl.BlockSpec(memory_space=pltpu.VMEM),
            pl.BlockSpec(memory_space=pltpu.SMEM),
        ],
        out_shape=[jax.ShapeDtypeStruct((16, sub, 1024), jnp.float32),
                   jax.ShapeDtypeStruct((1, sub, 1024), jnp.float32)],
    )(pre7, g3, shsr, par)
    safe = jnp.logical_not(jnp.any(aux[0] > 0.0))
    order = jnp.argsort(depths)
    attrs16 = attrs_u.reshape(16, Pn)[:, order]

    keys = pl.pallas_call(
        _keys_body,
        out_shape=jax.ShapeDtypeStruct((_KSLOT, sub, 1024), jnp.int32),
    )(attrs16.reshape(16, sub, 1024))
    keys = jnp.sort(keys.ravel())
    tile_arr = keys >> 13
    starts = jnp.searchsorted(
        tile_arr, jnp.arange(_NTILES + 1, dtype=jnp.int32)).astype(jnp.int32)
    idx = keys & (Pn - 1)
    ncap = (_KSLOT * Pn) // _CH
    binned = attrs16[:, idx].reshape(16, ncap, _CH).swapaxes(0, 1)

    def tiled_path():
        acc = _tile_composite(starts, binned)
        a = acc.reshape(_TGRID, _TGRID, _TILE, _TILE, 8)
        return a.transpose(1, 2, 0, 3, 4).reshape(_W * _H, 8)

    def dense_path():
        rgbd = attrs16[6:11].T
        rgbd = jnp.concatenate([rgbd, jnp.zeros((Pn, 3), jnp.float32)], axis=1)
        return _composite(attrs16[0:8], rgbd)

    acc = jax.lax.cond(safe, tiled_path, dense_path)
    accw = acc[:, 4:5]
    out_color = (acc[:, 0:3] + (1.0 - accw) * background[None, :]).reshape(_W, _H, 3)
    out_depth = acc[:, 3:4].reshape(_W, _H, 1)
    return out_color, out_depth, radii, visible
